# baseline (speedup 1.0000x reference)
import functools

import jax
import jax.numpy as jnp
from jax import lax
from jax.experimental import pallas as pl
from jax.experimental.pallas import tpu as pltpu
from jax.experimental.pallas import tpu_sc as plsc

N = 10000
E = 320000
D = 128
ALPHA_SLOPE = 0.2
RATE_MIX = 0.5

NW = 32
CHUNK = 128
NCH = 80
EP = NCH * CHUNK
E_PAD = EP * NW
N_ACC = N + 8
ROWS_PER_WRITER = 1000
NWRITERS = N // ROWS_PER_WRITER

_MESH = plsc.VectorSubcoreMesh(core_axis_name="c", subcore_axis_name="s")


@functools.partial(
    pl.kernel,
    mesh=_MESH,
    out_type=jax.ShapeDtypeStruct((2, N, D), jnp.float32),
    scratch_types=[
        pltpu.VMEM((NCH, CHUNK), jnp.int32),
        pltpu.VMEM((NCH, CHUNK), jnp.int32),
        pltpu.VMEM((CHUNK, D), jnp.float32),
        pltpu.VMEM_SHARED((N_ACC, D), jnp.float32),
        pltpu.SemaphoreType.DMA,
    ],
)
def _segsum_sc(table_hbm, src_hbm, dst_hbm, out_hbm,
               src_v, dst_v, rows_v, acc_sh, sem):
    cid = lax.axis_index("c")
    sid = lax.axis_index("s")
    wid = sid * 2 + cid

    zvec = jnp.zeros((16,), jnp.float32)

    def _zrow(r, carry):
        for k in range(D // 16):
            rows_v[r, pl.ds(k * 16, 16)] = zvec
        return carry

    lax.fori_loop(0, CHUNK, _zrow, 0)

    @pl.when(sid < NWRITERS)
    def _zero_acc():
        base = sid * ROWS_PER_WRITER
        for t in range(7):
            pltpu.sync_copy(rows_v, acc_sh.at[pl.ds(base + t * CHUNK, CHUNK)])
        pltpu.sync_copy(rows_v.at[pl.ds(0, 104)],
                        acc_sh.at[pl.ds(base + 896, 104)])

    plsc.subcore_barrier()

    pltpu.sync_copy(src_hbm.at[wid], src_v)
    pltpu.sync_copy(dst_hbm.at[wid], dst_v)

    def _body(j, carry):
        pltpu.async_copy(table_hbm.at[src_v.at[j]], rows_v, sem).wait()
        pltpu.sync_copy(rows_v, acc_sh.at[dst_v.at[j]], add=True)
        return carry

    lax.fori_loop(0, NCH, _body, 0)
    plsc.subcore_barrier()

    @pl.when(sid < NWRITERS)
    def _write_out():
        pltpu.sync_copy(
            acc_sh.at[pl.ds(sid * ROWS_PER_WRITER, ROWS_PER_WRITER)],
            out_hbm.at[cid, pl.ds(sid * ROWS_PER_WRITER, ROWS_PER_WRITER)])


def _segment_sum(table, edges):
    pad_dst = jnp.full((E_PAD - E,), N, jnp.int32)
    pad_src = jnp.zeros((E_PAD - E,), jnp.int32)
    dst = jnp.concatenate([edges[0], pad_dst]).reshape(NW, NCH, CHUNK)
    src = jnp.concatenate([edges[1], pad_src]).reshape(NW, NCH, CHUNK)
    return _segsum_sc(table, src, dst)


RB = 2000
NB = N // RB


def _mm_batched_body(x_ref, w_ref, o_ref):
    o_ref[...] = jnp.dot(x_ref[0], w_ref[0],
                         preferred_element_type=jnp.float32)[None]


def _support1(ufeas, gw1):
    return pl.pallas_call(
        _mm_batched_body,
        grid=(2, NB),
        in_specs=[
            pl.BlockSpec((1, RB, D), lambda i, b: (i, b, 0)),
            pl.BlockSpec((1, D, D), lambda i, b: (i, 0, 0)),
        ],
        out_specs=pl.BlockSpec((1, RB, D), lambda i, b: (i, b, 0)),
        out_shape=jax.ShapeDtypeStruct((2, N, D), jnp.float32),
    )(ufeas, gw1)


def _leaky(x):
    return jnp.where(x > 0, x, ALPHA_SLOPE * x)


def _stage_mid_body(p_ref, b_ref, w_ref, o_ref):
    agg = p_ref[0] + p_ref[1]
    h = _leaky(agg + b_ref[...])
    o_ref[...] = jnp.dot(h, w_ref[...], preferred_element_type=jnp.float32)


def _stage_mid(parts, b, w):
    return pl.pallas_call(
        _stage_mid_body,
        grid=(NB,),
        in_specs=[
            pl.BlockSpec((2, RB, D), lambda bk: (0, bk, 0)),
            pl.BlockSpec((D,), lambda bk: (0,)),
            pl.BlockSpec((D, D), lambda bk: (0, 0)),
        ],
        out_specs=pl.BlockSpec((RB, D), lambda bk: (bk, 0)),
        out_shape=jax.ShapeDtypeStruct((N, D), jnp.float32),
    )(parts, b, w)


def _stage_head_body(p_ref, gb_ref, uf_ref, wa_ref, wb_ref, ub_ref, o_ref):
    h = _leaky(p_ref[0] + p_ref[1] + gb_ref[...])
    out = (jnp.dot(h, wa_ref[...], preferred_element_type=jnp.float32)
           + jnp.dot(uf_ref[...], wb_ref[...], preferred_element_type=jnp.float32)
           + ub_ref[...])
    o_ref[...] = jnp.maximum(out, 0.0)


def _stage_head(parts, gb, ufea, uwa, uwb, ub):
    return pl.pallas_call(
        _stage_head_body,
        grid=(NB,),
        in_specs=[
            pl.BlockSpec((2, RB, D), lambda bk: (0, bk, 0)),
            pl.BlockSpec((D,), lambda bk: (0,)),
            pl.BlockSpec((RB, D), lambda bk: (bk, 0)),
            pl.BlockSpec((D, D), lambda bk: (0, 0)),
            pl.BlockSpec((D, D), lambda bk: (0, 0)),
            pl.BlockSpec((D,), lambda bk: (0,)),
        ],
        out_specs=pl.BlockSpec((RB, D), lambda bk: (bk, 0)),
        out_shape=jax.ShapeDtypeStruct((N, D), jnp.float32),
    )(parts, gb, ufea, uwa, uwb, ub)


def _combine_body(a_ref, b_ref, o_ref):
    o_ref[...] = RATE_MIX * a_ref[...] + (1.0 - RATE_MIX) * b_ref[...]


def _combine(r0, r1):
    return pl.pallas_call(
        _combine_body,
        grid=(NB,),
        in_specs=[
            pl.BlockSpec((RB, D), lambda bk: (bk, 0)),
            pl.BlockSpec((RB, D), lambda bk: (bk, 0)),
        ],
        out_specs=pl.BlockSpec((RB, D), lambda bk: (bk, 0)),
        out_shape=jax.ShapeDtypeStruct((N, D), jnp.float32),
    )(r0, r1)


def kernel(UFEAs, UVs, VUs, gw1, gb1, gw2, gb2, uw, ub):
    support1 = _support1(UFEAs, gw1)
    outs = []
    for i in range(2):
        p1 = _segment_sum(support1[i], VUs[i])
        support2 = _stage_mid(p1, gb1[i], gw2[i])
        p2 = _segment_sum(support2, UVs[i])
        r = _stage_head(p2, gb2[i], UFEAs[i],
                        uw[i, :D], uw[i, D:], ub[i])
        outs.append(r)
    return _combine(outs[0], outs[1])


# speedup vs baseline: 3.9254x; 3.9254x over previous
import functools

import jax
import jax.numpy as jnp
from jax import lax
from jax.experimental import pallas as pl
from jax.experimental.pallas import tpu as pltpu
from jax.experimental.pallas import tpu_sc as plsc

N = 10000
E = 320000
D = 128
ALPHA_SLOPE = 0.2
RATE_MIX = 0.5

NW = 32
CHUNK = 80
NCH = 125
EP = NCH * CHUNK
NH0 = 64
NH1 = NCH - NH0
ROWS_PER_WRITER = 1000
NWRITERS = N // ROWS_PER_WRITER

_MESH = plsc.VectorSubcoreMesh(core_axis_name="c", subcore_axis_name="s")


@functools.partial(
    pl.kernel,
    mesh=_MESH,
    out_type=jax.ShapeDtypeStruct((2, N, D), jnp.float32),
    scratch_types=[
        pltpu.VMEM((NH0, CHUNK), jnp.int32),
        pltpu.VMEM((NH0, CHUNK), jnp.int32),
        pltpu.VMEM((CHUNK, D), jnp.float32),
        pltpu.VMEM((CHUNK, D), jnp.float32),
        pltpu.VMEM_SHARED((N, D), jnp.float32),
        pltpu.SemaphoreType.DMA,
        pltpu.SemaphoreType.DMA,
    ],
)
def _segsum_sc(table_hbm, src_hbm, dst_hbm, out_hbm,
               src_v, dst_v, rows_v, rows1_v, acc_sh, sem, sem1):
    cid = lax.axis_index("c")
    sid = lax.axis_index("s")
    wid = sid * 2 + cid

    zvec = jnp.zeros((16,), jnp.float32)

    def _zrow(r, carry):
        for k in range(D // 16):
            rows_v[r, pl.ds(k * 16, 16)] = zvec
        return carry

    lax.fori_loop(0, CHUNK, _zrow, 0)

    @pl.when(sid < NWRITERS)
    def _zero_acc():
        base = sid * ROWS_PER_WRITER
        for t in range(ROWS_PER_WRITER // CHUNK):
            pltpu.sync_copy(rows_v, acc_sh.at[pl.ds(base + t * CHUNK, CHUNK)])
        pltpu.sync_copy(rows_v.at[pl.ds(0, 40)],
                        acc_sh.at[pl.ds(base + 960, 40)])

    plsc.subcore_barrier()

    def _wait_rows(buf, s):
        pltpu.make_async_copy(table_hbm.at[pl.ds(0, CHUNK)], buf, s).wait()

    for h, hn in ((0, NH0), (1, NH1)):
        pltpu.sync_copy(src_hbm.at[wid, pl.ds(h * NH0, hn)],
                        src_v.at[pl.ds(0, hn)])
        pltpu.sync_copy(dst_hbm.at[wid, pl.ds(h * NH0, hn)],
                        dst_v.at[pl.ds(0, hn)])
        pltpu.async_copy(table_hbm.at[src_v.at[0]], rows_v, sem)

        def _pair(p, carry):
            j0 = 2 * p
            pltpu.async_copy(table_hbm.at[src_v.at[j0 + 1]], rows1_v, sem1)
            _wait_rows(rows_v, sem)
            pltpu.sync_copy(rows_v, acc_sh.at[dst_v.at[j0]], add=True)
            pltpu.async_copy(table_hbm.at[src_v.at[j0 + 2]], rows_v, sem)
            _wait_rows(rows1_v, sem1)
            pltpu.sync_copy(rows1_v, acc_sh.at[dst_v.at[j0 + 1]], add=True)
            return carry

        npairs = (hn - 2) // 2 if hn % 2 == 0 else (hn - 1) // 2
        lax.fori_loop(0, npairs, _pair, 0)
        if hn % 2 == 0:
            pltpu.async_copy(table_hbm.at[src_v.at[hn - 1]], rows1_v, sem1)
            _wait_rows(rows_v, sem)
            pltpu.sync_copy(rows_v, acc_sh.at[dst_v.at[hn - 2]], add=True)
            _wait_rows(rows1_v, sem1)
            pltpu.sync_copy(rows1_v, acc_sh.at[dst_v.at[hn - 1]], add=True)
        else:
            _wait_rows(rows_v, sem)
            pltpu.sync_copy(rows_v, acc_sh.at[dst_v.at[hn - 1]], add=True)
    plsc.subcore_barrier()

    @pl.when(sid < NWRITERS)
    def _write_out():
        pltpu.sync_copy(
            acc_sh.at[pl.ds(sid * ROWS_PER_WRITER, ROWS_PER_WRITER)],
            out_hbm.at[cid, pl.ds(sid * ROWS_PER_WRITER, ROWS_PER_WRITER)])


def _segment_sum(table, edges):
    dst = edges[0].reshape(NW, NCH, CHUNK)
    src = edges[1].reshape(NW, NCH, CHUNK)
    return _segsum_sc(table, src, dst)


RB = 2000
NB = N // RB


def _mm_batched_body(x_ref, w_ref, o_ref):
    o_ref[...] = jnp.dot(x_ref[0], w_ref[0],
                         preferred_element_type=jnp.float32)[None]


def _support1(ufeas, gw1):
    return pl.pallas_call(
        _mm_batched_body,
        grid=(2, NB),
        in_specs=[
            pl.BlockSpec((1, RB, D), lambda i, b: (i, b, 0)),
            pl.BlockSpec((1, D, D), lambda i, b: (i, 0, 0)),
        ],
        out_specs=pl.BlockSpec((1, RB, D), lambda i, b: (i, b, 0)),
        out_shape=jax.ShapeDtypeStruct((2, N, D), jnp.float32),
    )(ufeas, gw1)


def _leaky(x):
    return jnp.where(x > 0, x, ALPHA_SLOPE * x)


def _stage_mid_body(p_ref, b_ref, w_ref, o_ref):
    agg = p_ref[0] + p_ref[1]
    h = _leaky(agg + b_ref[...])
    o_ref[...] = jnp.dot(h, w_ref[...], preferred_element_type=jnp.float32)


def _stage_mid(parts, b, w):
    return pl.pallas_call(
        _stage_mid_body,
        grid=(NB,),
        in_specs=[
            pl.BlockSpec((2, RB, D), lambda bk: (0, bk, 0)),
            pl.BlockSpec((D,), lambda bk: (0,)),
            pl.BlockSpec((D, D), lambda bk: (0, 0)),
        ],
        out_specs=pl.BlockSpec((RB, D), lambda bk: (bk, 0)),
        out_shape=jax.ShapeDtypeStruct((N, D), jnp.float32),
    )(parts, b, w)


def _stage_head_body(p_ref, gb_ref, uf_ref, wa_ref, wb_ref, ub_ref, o_ref):
    h = _leaky(p_ref[0] + p_ref[1] + gb_ref[...])
    out = (jnp.dot(h, wa_ref[...], preferred_element_type=jnp.float32)
           + jnp.dot(uf_ref[...], wb_ref[...], preferred_element_type=jnp.float32)
           + ub_ref[...])
    o_ref[...] = jnp.maximum(out, 0.0)


def _stage_head(parts, gb, ufea, uwa, uwb, ub):
    return pl.pallas_call(
        _stage_head_body,
        grid=(NB,),
        in_specs=[
            pl.BlockSpec((2, RB, D), lambda bk: (0, bk, 0)),
            pl.BlockSpec((D,), lambda bk: (0,)),
            pl.BlockSpec((RB, D), lambda bk: (bk, 0)),
            pl.BlockSpec((D, D), lambda bk: (0, 0)),
            pl.BlockSpec((D, D), lambda bk: (0, 0)),
            pl.BlockSpec((D,), lambda bk: (0,)),
        ],
        out_specs=pl.BlockSpec((RB, D), lambda bk: (bk, 0)),
        out_shape=jax.ShapeDtypeStruct((N, D), jnp.float32),
    )(parts, gb, ufea, uwa, uwb, ub)


def _combine_body(a_ref, b_ref, o_ref):
    o_ref[...] = RATE_MIX * a_ref[...] + (1.0 - RATE_MIX) * b_ref[...]


def _combine(r0, r1):
    return pl.pallas_call(
        _combine_body,
        grid=(NB,),
        in_specs=[
            pl.BlockSpec((RB, D), lambda bk: (bk, 0)),
            pl.BlockSpec((RB, D), lambda bk: (bk, 0)),
        ],
        out_specs=pl.BlockSpec((RB, D), lambda bk: (bk, 0)),
        out_shape=jax.ShapeDtypeStruct((N, D), jnp.float32),
    )(r0, r1)


def kernel(UFEAs, UVs, VUs, gw1, gb1, gw2, gb2, uw, ub):
    support1 = _support1(UFEAs, gw1)
    outs = []
    for i in range(2):
        p1 = _segment_sum(support1[i], VUs[i])
        support2 = _stage_mid(p1, gb1[i], gw2[i])
        p2 = _segment_sum(support2, UVs[i])
        r = _stage_head(p2, gb2[i], UFEAs[i],
                        uw[i, :D], uw[i, D:], ub[i])
        outs.append(r)
    return _combine(outs[0], outs[1])
